# Initial kernel scaffold; baseline (speedup 1.0000x reference)
import jax
import jax.numpy as jnp
from jax.experimental import pallas as pl


def kernel(x1, edge_attr1, x2, edge_attr2, params, edge_index1, edge_index2):
    raise NotImplementedError("write your pallas kernel here")


# speedup vs baseline: 1.9416x; 1.9416x over previous
import jax
import jax.numpy as jnp
from jax.experimental import pallas as pl

K_NEAR = 10


def _gat(p, x, src, dst, n):
    h = x @ p["W"]
    a_src = h @ p["att_src"]
    a_dst = h @ p["att_dst"]
    alpha = jax.nn.leaky_relu(a_src[src] + a_dst[dst], negative_slope=0.2)
    ex = jnp.exp(alpha)
    denom = jax.ops.segment_sum(ex, dst, num_segments=n)
    coef = ex / denom[dst]
    out = jax.ops.segment_sum(coef[:, None] * h[src], dst, num_segments=n)
    return out + p["bias"]


def _each_graph(params, x, edge_index):
    n = x.shape[0]
    loop = jnp.arange(n, dtype=edge_index.dtype)
    src = jnp.concatenate([edge_index[0], loop])
    dst = jnp.concatenate([edge_index[1], loop])

    enc = params["enc1"]
    xs = [x]
    for p in enc["layers"][:5]:
        xs.append(_gat(p, xs[-1], src, dst, n))
    xcat1 = jnp.concatenate(xs, axis=-1)[:, :321]
    emb1 = xcat1 @ enc["conv_w"][:, 0, :].T + enc["conv_b"]

    enc = params["enc2"]
    ys = [x]
    for p in enc["layers"][:2]:
        ys.append(_gat(p, ys[-1], src, dst, n))
    xcat2 = jnp.concatenate(ys, axis=-1)[:, :25]
    emb2 = xcat2 @ enc["conv_w"][:, 0, :].T + enc["conv_b"]

    return jnp.concatenate([emb2, emb1], axis=1)


def kernel(x1, edge_attr1, x2, edge_attr2, params, edge_index1, edge_index2):
    n1 = x1.shape[0]
    pred1 = _each_graph(params, x1, edge_index1)
    pred2 = _each_graph(params, x2, edge_index2)

    a = x1[:, 0:3]
    b = x2[:, 0:3]
    aa = jnp.sum(a * a, axis=1)[:, None]
    bb = jnp.sum(b * b, axis=1)[None, :]
    d2 = aa + bb - 2.0 * (a @ b.T)
    _, nearest = jax.lax.top_k(-d2, K_NEAR)
    cols = jnp.sort(nearest, axis=1).reshape(-1)
    rows = jnp.repeat(jnp.arange(n1), K_NEAR)
    diff = jnp.abs(pred1[rows] - pred2[cols])

    mlp = params["mlp"]
    W1 = mlp["W1"][0]
    W2 = mlp["W2"][:, 0]
    t = jnp.zeros_like(diff) + mlp["b2"][0]
    for j in range(32):
        t = t + W2[j] * jax.nn.relu(diff * W1[j] + mlp["b1"][j])
    M = params["m2_w"] @ params["m_w"]
    c = params["m2_w"] @ params["m_b"] + params["m2_b"]
    return t @ M.T + c


# speedup vs baseline: 2.1655x; 1.1153x over previous
import jax
import jax.numpy as jnp
from jax import lax
from jax.experimental import pallas as pl

K_NEAR = 10


def _knn_body(q_ref, bt_ref, out_ref):
    q = q_ref[...]
    bt = bt_ref[...]
    aa = jnp.sum(q * q, axis=1, keepdims=True)
    bb = jnp.sum(bt * bt, axis=0, keepdims=True)
    d2 = aa + bb - 2.0 * jnp.dot(q, bt, preferred_element_type=jnp.float32)
    br, n2 = d2.shape
    iota = lax.broadcasted_iota(jnp.int32, (br, n2), 1)
    BIGI = jnp.int32(2**30)
    INF = jnp.float32(3e38)
    sels = []
    for _ in range(K_NEAR):
        m = jnp.min(d2, axis=1, keepdims=True)
        cand = jnp.where(d2 <= m, iota, BIGI)
        sel = jnp.min(cand, axis=1, keepdims=True)
        sels.append(sel)
        d2 = jnp.where(iota == sel, INF, d2)
    for r in range(K_NEAR):
        for i in range(r % 2, K_NEAR - 1, 2):
            lo = jnp.minimum(sels[i], sels[i + 1])
            hi = jnp.maximum(sels[i], sels[i + 1])
            sels[i], sels[i + 1] = lo, hi
    lane = lax.broadcasted_iota(jnp.int32, (br, 128), 1)
    res = jnp.zeros((br, 128), jnp.int32)
    for k in range(K_NEAR):
        res = jnp.where(lane == k, sels[k], res)
    out_ref[...] = res


def _knn(a, b, br=200):
    n1, n2 = a.shape[0], b.shape[0]
    qpad = jnp.zeros((n1, 8), jnp.float32).at[:, :3].set(a)
    btpad = jnp.zeros((8, n2), jnp.float32).at[:3, :].set(b.T)
    return pl.pallas_call(
        _knn_body,
        grid=(n1 // br,),
        in_specs=[
            pl.BlockSpec((br, 8), lambda i: (i, 0)),
            pl.BlockSpec((8, n2), lambda i: (0, 0)),
        ],
        out_specs=pl.BlockSpec((br, 128), lambda i: (i, 0)),
        out_shape=jax.ShapeDtypeStruct((n1, 128), jnp.int32),
    )(qpad, btpad)


def _gat(p, x, src, dst, n):
    h = x @ p["W"]
    a_src = h @ p["att_src"]
    a_dst = h @ p["att_dst"]
    alpha = jax.nn.leaky_relu(a_src[src] + a_dst[dst], negative_slope=0.2)
    ex = jnp.exp(alpha)
    denom = jax.ops.segment_sum(ex, dst, num_segments=n)
    coef = ex / denom[dst]
    out = jax.ops.segment_sum(coef[:, None] * h[src], dst, num_segments=n)
    return out + p["bias"]


def _each_graph(params, x, edge_index):
    n = x.shape[0]
    loop = jnp.arange(n, dtype=edge_index.dtype)
    src = jnp.concatenate([edge_index[0], loop])
    dst = jnp.concatenate([edge_index[1], loop])

    enc = params["enc1"]
    xs = [x]
    for p in enc["layers"][:5]:
        xs.append(_gat(p, xs[-1], src, dst, n))
    xcat1 = jnp.concatenate(xs, axis=-1)[:, :321]
    emb1 = xcat1 @ enc["conv_w"][:, 0, :].T + enc["conv_b"]

    enc = params["enc2"]
    ys = [x]
    for p in enc["layers"][:2]:
        ys.append(_gat(p, ys[-1], src, dst, n))
    xcat2 = jnp.concatenate(ys, axis=-1)[:, :25]
    emb2 = xcat2 @ enc["conv_w"][:, 0, :].T + enc["conv_b"]

    return jnp.concatenate([emb2, emb1], axis=1)


def kernel(x1, edge_attr1, x2, edge_attr2, params, edge_index1, edge_index2):
    n1 = x1.shape[0]
    pred1 = _each_graph(params, x1, edge_index1)
    pred2 = _each_graph(params, x2, edge_index2)

    cols = _knn(x1[:, 0:3], x2[:, 0:3])[:, :K_NEAR].reshape(-1)
    rows = jnp.repeat(jnp.arange(n1), K_NEAR)
    diff = jnp.abs(pred1[rows] - pred2[cols])

    mlp = params["mlp"]
    W1 = mlp["W1"][0]
    W2 = mlp["W2"][:, 0]
    t = jnp.zeros_like(diff) + mlp["b2"][0]
    for j in range(32):
        t = t + W2[j] * jax.nn.relu(diff * W1[j] + mlp["b1"][j])
    M = params["m2_w"] @ params["m_w"]
    c = params["m2_w"] @ params["m_b"] + params["m2_b"]
    return t @ M.T + c


# speedup vs baseline: 26.6352x; 12.2997x over previous
import functools

import jax
import jax.numpy as jnp
from jax import lax
from jax.experimental import pallas as pl
from jax.experimental.pallas import tpu as pltpu
from jax.experimental.pallas import tpu_sc as plsc

K_NEAR = 10
NW = 32
F32 = jnp.float32


def _knn_body(q_ref, bt_ref, out_ref):
    q = q_ref[...]
    bt = bt_ref[...]
    aa = jnp.sum(q * q, axis=1, keepdims=True)
    bb = jnp.sum(bt * bt, axis=0, keepdims=True)
    d2 = aa + bb - 2.0 * jnp.dot(q, bt, preferred_element_type=F32)
    d2 = jnp.sqrt(jnp.maximum(d2, 0.0))
    br, n2 = d2.shape
    iota = lax.broadcasted_iota(jnp.int32, (br, n2), 1)
    BIGI = jnp.int32(2**30)
    INF = jnp.float32(3e38)
    sels = []
    for _ in range(K_NEAR):
        m = jnp.min(d2, axis=1, keepdims=True)
        cand = jnp.where(d2 <= m, iota, BIGI)
        sel = jnp.min(cand, axis=1, keepdims=True)
        sels.append(sel)
        d2 = jnp.where(iota == sel, INF, d2)
    for r in range(K_NEAR):
        for i in range(r % 2, K_NEAR - 1, 2):
            lo = jnp.minimum(sels[i], sels[i + 1])
            hi = jnp.maximum(sels[i], sels[i + 1])
            sels[i], sels[i + 1] = lo, hi
    lane = lax.broadcasted_iota(jnp.int32, (br, 128), 1)
    res = jnp.zeros((br, 128), jnp.int32)
    for k in range(K_NEAR):
        res = jnp.where(lane == k, sels[k], res)
    out_ref[...] = res


def _knn(a, b, br=200):
    n1, n2 = a.shape[0], b.shape[0]
    qpad = jnp.zeros((n1, 8), F32).at[:, :3].set(a)
    btpad = jnp.zeros((8, n2), F32).at[:3, :].set(b.T)
    return pl.pallas_call(
        _knn_body,
        grid=(n1 // br,),
        in_specs=[
            pl.BlockSpec((br, 8), lambda i: (i, 0)),
            pl.BlockSpec((8, n2), lambda i: (0, 0)),
        ],
        out_specs=pl.BlockSpec((br, 128), lambda i: (i, 0)),
        out_shape=jax.ShapeDtypeStruct((n1, 128), jnp.int32),
    )(qpad, btpad)


def _dense_first_body(x_ref, w_ref, attm_ref, h_ref, a2_ref):
    x = x_ref[...]
    h = jnp.dot(x, w_ref[...], preferred_element_type=F32)
    h_ref[...] = h
    a2_ref[...] = jnp.dot(h, attm_ref[...], preferred_element_type=F32)


def _dense_first(x_pad, w_pad, attm, bn=1280):
    n_pad = x_pad.shape[0]
    fin = x_pad.shape[1]
    fpad = w_pad.shape[1]
    return pl.pallas_call(
        _dense_first_body,
        grid=(n_pad // bn,),
        in_specs=[
            pl.BlockSpec((bn, fin), lambda i: (i, 0)),
            pl.BlockSpec(w_pad.shape, lambda i: (0, 0)),
            pl.BlockSpec(attm.shape, lambda i: (0, 0)),
        ],
        out_specs=(pl.BlockSpec((bn, fpad), lambda i: (i, 0)),
                   pl.BlockSpec((bn, 8), lambda i: (i, 0))),
        out_shape=(jax.ShapeDtypeStruct((n_pad, fpad), F32),
                   jax.ShapeDtypeStruct((n_pad, 8), F32)),
    )(x_pad, w_pad, attm)


def _dense_mid_body(u_ref, dent_ref, b_ref, w_ref, attm_ref,
                    xl_ref, h_ref, a2_ref):
    den = jnp.sum(dent_ref[...], axis=1, keepdims=True)
    xl = (u_ref[0] + u_ref[1]) / jnp.maximum(den, 1e-30) + b_ref[...]
    xl_ref[...] = xl
    h = jnp.dot(xl, w_ref[...], preferred_element_type=F32)
    h_ref[...] = h
    a2_ref[...] = jnp.dot(h, attm_ref[...], preferred_element_type=F32)


def _dense_mid(u, dent, bias_row, w_pad, attm, bn=1280):
    n_pad = u.shape[1]
    fprev = u.shape[2]
    fpad = w_pad.shape[1]
    nw = dent.shape[1]
    return pl.pallas_call(
        _dense_mid_body,
        grid=(n_pad // bn,),
        in_specs=[
            pl.BlockSpec((2, bn, fprev), lambda i: (0, i, 0)),
            pl.BlockSpec((bn, nw), lambda i: (i, 0)),
            pl.BlockSpec((1, fprev), lambda i: (0, 0)),
            pl.BlockSpec(w_pad.shape, lambda i: (0, 0)),
            pl.BlockSpec(attm.shape, lambda i: (0, 0)),
        ],
        out_specs=(pl.BlockSpec((bn, fprev), lambda i: (i, 0)),
                   pl.BlockSpec((bn, fpad), lambda i: (i, 0)),
                   pl.BlockSpec((bn, 8), lambda i: (i, 0))),
        out_shape=(jax.ShapeDtypeStruct((n_pad, fprev), F32),
                   jax.ShapeDtypeStruct((n_pad, fpad), F32),
                   jax.ShapeDtypeStruct((n_pad, 8), F32)),
    )(u, dent, bias_row, w_pad, attm)


def _gat_agg(packed, asrc, adst, h):
    n_pad, fpad = h.shape
    nchunks = packed.shape[1]
    rpt = n_pad // 16
    mesh = plsc.VectorSubcoreMesh(core_axis_name="c", subcore_axis_name="s")

    @functools.partial(
        pl.kernel,
        out_type=(jax.ShapeDtypeStruct((2, n_pad, fpad), F32),
                  jax.ShapeDtypeStruct((NW, n_pad), F32)),
        mesh=mesh,
        compiler_params=pltpu.CompilerParams(needs_layout_passes=False, use_tc_tiling_on_sc=False),
        scratch_types=[
            pltpu.VMEM((nchunks, 128), jnp.int32),
            pltpu.VMEM((nchunks, 128), jnp.int32),
            pltpu.VMEM((n_pad,), F32),
            pltpu.VMEM((n_pad,), F32),
            pltpu.VMEM((n_pad,), F32),
            pltpu.VMEM((128, fpad), F32),
            pltpu.VMEM((128,), F32),
            pltpu.VMEM((rpt // 8, fpad), F32),
            pltpu.VMEM_SHARED((n_pad, fpad), F32),
            pltpu.SemaphoreType.DMA,
        ],
    )
    def k(packed_hbm, asrc_hbm, adst_hbm, h_hbm, u_out, den_out,
          src_v, dst_v, asrc_v, adst_v, den_loc, rowbuf, exbuf, zbuf,
          u_sh, sem):
        c = lax.axis_index("c")
        s = lax.axis_index("s")
        wid = s * 2 + c
        zero16 = jnp.zeros((16,), F32)
        zrows = rpt // 8

        pltpu.sync_copy(packed_hbm.at[wid], src_v)
        pltpu.sync_copy(asrc_hbm, asrc_v)
        pltpu.sync_copy(adst_hbm, adst_v)

        def unpk(i, carry):
            for j in range(8):
                v = src_v[i, pl.ds(j * 16, 16)]
                src_v[i, pl.ds(j * 16, 16)] = v & jnp.int32(0xFFFF)
                dst_v[i, pl.ds(j * 16, 16)] = lax.shift_right_logical(v, 16)
            return carry
        lax.fori_loop(0, nchunks, unpk, 0)

        def zdl(i, carry):
            den_loc[pl.ds(i * 16, 16)] = zero16
            return carry
        lax.fori_loop(0, n_pad // 16, zdl, 0)

        def zzb(i, carry):
            for cc in range(fpad // 16):
                zbuf[i, pl.ds(cc * 16, 16)] = zero16
            return carry
        lax.fori_loop(0, zrows, zzb, 0)
        for kk in range(8):
            pltpu.sync_copy(zbuf, u_sh.at[pl.ds(s * rpt + kk * zrows, zrows)])
        plsc.subcore_barrier()

        lane16 = lax.broadcasted_iota(jnp.int32, (16,), 0)

        def chunk(i, carry):
            pltpu.async_copy(h_hbm.at[src_v.at[i]], rowbuf, sem).wait()

            def grp(j, carry2):
                s16 = src_v[i, pl.ds(j * 16, 16)]
                d16 = dst_v[i, pl.ds(j * 16, 16)]
                av = (plsc.load_gather(asrc_v, [s16])
                      + plsc.load_gather(adst_v, [d16]))
                al = jnp.where(av >= 0, av, jnp.float32(0.2) * av)
                ex = jnp.exp(al)
                exbuf[pl.ds(j * 16, 16)] = ex
                for l in range(16):
                    plsc.addupdate_scatter(den_loc, [d16], ex,
                                           mask=lane16 == l)
                return carry2
            lax.fori_loop(0, 8, grp, 0)

            def rmul(j, carry2):
                ex16 = exbuf[pl.ds(j * 16, 16)]
                for l in range(16):
                    ce = ex16[l]
                    for cc in range(fpad // 16):
                        e = j * 16 + l
                        rowbuf[e, pl.ds(cc * 16, 16)] = (
                            rowbuf[e, pl.ds(cc * 16, 16)] * ce)
                return carry2
            lax.fori_loop(0, 8, rmul, 0)

            pltpu.sync_copy(rowbuf, u_sh.at[dst_v.at[i]], add=True)
            return carry
        lax.fori_loop(0, nchunks, chunk, 0)
        pltpu.sync_copy(den_loc, den_out.at[wid])
        plsc.subcore_barrier()

        pltpu.sync_copy(u_sh.at[pl.ds(s * rpt, rpt)],
                        u_out.at[c, pl.ds(s * rpt, rpt)])

    return k(packed, asrc, adst, h)


def _match_diff(ridx2d, cidx2d, pred1, pred2):
    nchunks = ridx2d.shape[1]
    ppw = nchunks * 128
    p_pad = NW * ppw
    mesh = plsc.VectorSubcoreMesh(core_axis_name="c", subcore_axis_name="s")

    @functools.partial(
        pl.kernel,
        out_type=jax.ShapeDtypeStruct((p_pad, 32), F32),
        mesh=mesh,
        compiler_params=pltpu.CompilerParams(needs_layout_passes=False, use_tc_tiling_on_sc=False),
        scratch_types=[
            pltpu.VMEM((nchunks, 128), jnp.int32),
            pltpu.VMEM((nchunks, 128), jnp.int32),
            pltpu.VMEM((128, 32), F32),
            pltpu.VMEM((128, 32), F32),
            pltpu.SemaphoreType.DMA,
        ],
    )
    def k(ridx_hbm, cidx_hbm, p1_hbm, p2_hbm, out_hbm, r_v, c_v, b1, b2, sem):
        c = lax.axis_index("c")
        s = lax.axis_index("s")
        wid = s * 2 + c
        pltpu.sync_copy(ridx_hbm.at[wid], r_v)
        pltpu.sync_copy(cidx_hbm.at[wid], c_v)

        def chunk(i, carry):
            pltpu.async_copy(p1_hbm.at[r_v.at[i]], b1, sem).wait()
            pltpu.async_copy(p2_hbm.at[c_v.at[i]], b2, sem).wait()

            def rm(e, carry2):
                for cc in range(2):
                    d = (b1[e, pl.ds(cc * 16, 16)]
                         - b2[e, pl.ds(cc * 16, 16)])
                    b1[e, pl.ds(cc * 16, 16)] = jnp.abs(d)
                return carry2
            lax.fori_loop(0, 128, rm, 0)
            pltpu.sync_copy(b1, out_hbm.at[pl.ds(wid * ppw + i * 128, 128)])
            return carry
        lax.fori_loop(0, nchunks, chunk, 0)

    return k(ridx2d, cidx2d, pred1, pred2)


def _project_body(x_ref, a1_ref, a2_ref, a3_ref, a4_ref, u5_ref, den5_ref,
                  b5_ref, g1_ref, ug2_ref, deng2_ref, bg2_ref,
                  wa_ref, wb_ref, cvec_ref, out_ref):
    den5 = jnp.sum(den5_ref[...], axis=1, keepdims=True)
    a5 = (u5_ref[0] + u5_ref[1]) / jnp.maximum(den5, 1e-30) + b5_ref[...]
    deng2 = jnp.sum(deng2_ref[...], axis=1, keepdims=True)
    g2 = (ug2_ref[0] + ug2_ref[1]) / jnp.maximum(deng2, 1e-30) + bg2_ref[...]
    wa = wa_ref[...]
    wb = wb_ref[...]
    x = x_ref[...]
    emb1 = (jnp.dot(x, wa[0:16], preferred_element_type=F32)
            + jnp.dot(a1_ref[...], wa[16:80], preferred_element_type=F32)
            + jnp.dot(a2_ref[...], wa[80:144], preferred_element_type=F32)
            + jnp.dot(a3_ref[...], wa[144:208], preferred_element_type=F32)
            + jnp.dot(a4_ref[...], wa[208:272], preferred_element_type=F32)
            + jnp.dot(a5, wa[272:336], preferred_element_type=F32))
    emb2 = (jnp.dot(x, wb[0:16], preferred_element_type=F32)
            + jnp.dot(g1_ref[...], wb[16:32], preferred_element_type=F32)
            + jnp.dot(g2, wb[32:48], preferred_element_type=F32))
    out_ref[...] = jnp.concatenate([emb2, emb1], axis=1) + cvec_ref[...]


def _project(x_pad, a_list, u5, den5t, b5row, g1, ug2, deng2t, bg2row,
             wa, wb, cvec, bn=1280):
    n_pad = x_pad.shape[0]
    nw = den5t.shape[1]
    full = lambda arr: pl.BlockSpec(arr.shape, lambda i: tuple(0 for _ in arr.shape))
    row2 = lambda f: pl.BlockSpec((bn, f), lambda i: (i, 0))
    return pl.pallas_call(
        _project_body,
        grid=(n_pad // bn,),
        in_specs=[
            row2(16), row2(64), row2(64), row2(64), row2(64),
            pl.BlockSpec((2, bn, 64), lambda i: (0, i, 0)),
            row2(nw), full(b5row), row2(16),
            pl.BlockSpec((2, bn, 16), lambda i: (0, i, 0)),
            row2(nw), full(bg2row), full(wa), full(wb), full(cvec),
        ],
        out_specs=row2(32),
        out_shape=jax.ShapeDtypeStruct((n_pad, 32), F32),
    )(x_pad, a_list[0], a_list[1], a_list[2], a_list[3], u5, den5t,
      b5row, g1, ug2, deng2t, bg2row, wa, wb, cvec)


def _mlp_body(d_ref, w1_ref, b1_ref, w2_ref, mt_ref, c_ref, out_ref):
    d = d_ref[...]
    acc = jnp.zeros_like(d)
    for j in range(32):
        acc = acc + w2_ref[0, j] * jnp.maximum(
            d * w1_ref[0, j] + b1_ref[0, j], 0.0)
    out_ref[...] = (jnp.dot(acc, mt_ref[...], preferred_element_type=F32)
                    + c_ref[...])


def _mlp(diff, w1, b1, w2, mt, cfold, bp=2048):
    p_pad = diff.shape[0]
    return pl.pallas_call(
        _mlp_body,
        grid=(p_pad // bp,),
        in_specs=[
            pl.BlockSpec((bp, 32), lambda i: (i, 0)),
            pl.BlockSpec(memory_space=pltpu.SMEM),
            pl.BlockSpec(memory_space=pltpu.SMEM),
            pl.BlockSpec(memory_space=pltpu.SMEM),
            pl.BlockSpec((32, 8), lambda i: (0, 0)),
            pl.BlockSpec((1, 8), lambda i: (0, 0)),
        ],
        out_specs=pl.BlockSpec((bp, 8), lambda i: (i, 0)),
        out_shape=jax.ShapeDtypeStruct((p_pad, 8), F32),
    )(diff, w1, b1, w2, mt, cfold)


def _pad_w(w, fin_pad, fout_pad):
    return jnp.zeros((fin_pad, fout_pad), F32).at[:w.shape[0], :w.shape[1]].set(w)


def _attm(p, fpad):
    m = jnp.zeros((fpad, 8), F32)
    m = m.at[:p["att_src"].shape[0], 0].set(p["att_src"])
    m = m.at[:p["att_dst"].shape[0], 1].set(p["att_dst"])
    return m


def _bias_row(b, fpad):
    return jnp.zeros((1, fpad), F32).at[0, :b.shape[0]].set(b)


def _dent(den, n_pad):
    return den.T


def _fin_pad(f):
    return max(16, -(-f // 16) * 16)


def _run_graph(params, x, edge_index, n):
    n_pad = 10240 if n == 10000 else -(-n // 256) * 256
    e_tot = edge_index.shape[1] + n
    e_pad = -(-e_tot // (NW * 128)) * (NW * 128)
    dummy = n

    loop = jnp.arange(n, dtype=edge_index.dtype)
    fill = jnp.full((e_pad - e_tot,), dummy, jnp.int32)
    src_all = jnp.concatenate([edge_index[0], loop, fill])
    dst_all = jnp.concatenate([edge_index[1], loop, fill])
    packed = (src_all | (dst_all << 16)).reshape(NW, -1, 128)

    x_pad = jnp.zeros((n_pad, 16), F32).at[:n, :x.shape[1]].set(x)

    def run_enc(layers, nlayers, fpad):
        feats = []
        u = den = None
        for li in range(nlayers):
            p = layers[li]
            w = _pad_w(p["W"], _fin_pad(p["W"].shape[0]) if li else 16, fpad)
            am = _attm(p, fpad)
            if li == 0:
                h, a2 = _dense_first(x_pad, w, am)
            else:
                bias = _bias_row(layers[li - 1]["bias"], fpad)
                xl, h, a2 = _dense_mid(u, _dent(den, n_pad), bias, w, am)
                feats.append(xl)
            asrc = a2[:, 0]
            adst = a2[:, 1]
            u, den = _gat_agg(packed, asrc, adst, h)
        return feats, u, den

    feats1, u5, den5 = run_enc(params["enc1"]["layers"], 5, 64)
    feats2, ug2, deng2 = run_enc(params["enc2"]["layers"], 2, 16)

    cw1 = params["enc1"]["conv_w"][:, 0, :]
    cw2 = params["enc2"]["conv_w"][:, 0, :]
    wa = jnp.zeros((336, 16), F32)
    wa = wa.at[0:10].set(cw1[:, 0:10].T)
    for i in range(4):
        wa = wa.at[16 + 64 * i:16 + 64 * i + 64].set(
            cw1[:, 10 + 64 * i:74 + 64 * i].T)
    wa = wa.at[272:272 + 55].set(cw1[:, 266:321].T)
    wb = jnp.zeros((48, 16), F32)
    wb = wb.at[0:10].set(cw2[:, 0:10].T)
    wb = wb.at[16:24].set(cw2[:, 10:18].T)
    wb = wb.at[32:39].set(cw2[:, 18:25].T)
    cvec = jnp.concatenate(
        [params["enc2"]["conv_b"], params["enc1"]["conv_b"]])[None, :]

    b5row = _bias_row(params["enc1"]["layers"][4]["bias"], 64)
    bg2row = _bias_row(params["enc2"]["layers"][1]["bias"], 16)
    return _project(x_pad, feats1, u5, _dent(den5, n_pad), b5row,
                    feats2[0], ug2, _dent(deng2, n_pad), bg2row,
                    wa, wb, cvec)


def kernel(x1, edge_attr1, x2, edge_attr2, params, edge_index1, edge_index2):
    n1 = x1.shape[0]
    n2 = x2.shape[0]
    pred1 = _run_graph(params, x1, edge_index1, n1)
    pred2 = _run_graph(params, x2, edge_index2, n2)

    cols = _knn(x1[:, 0:3], x2[:, 0:3])[:, :K_NEAR].reshape(-1)
    p = n1 * K_NEAR
    p_pad = -(-p // (NW * 128)) * (NW * 128)
    rows = jnp.repeat(jnp.arange(n1, dtype=jnp.int32), K_NEAR)
    ridx2d = jnp.concatenate(
        [rows, jnp.zeros((p_pad - p,), jnp.int32)]).reshape(NW, -1, 128)
    cidx2d = jnp.concatenate(
        [cols, jnp.zeros((p_pad - p,), jnp.int32)]).reshape(NW, -1, 128)

    diff = _match_diff(ridx2d, cidx2d, pred1, pred2)

    mlp = params["mlp"]
    w1 = mlp["W1"][0][None, :]
    b1 = mlp["b1"][None, :]
    w2 = mlp["W2"][:, 0][None, :]
    m = params["m2_w"] @ params["m_w"]
    mt = jnp.zeros((32, 8), F32).at[:, :2].set(m.T)
    c = params["m2_w"] @ params["m_b"] + params["m2_b"]
    b2 = mlp["b2"][0]
    cfold = jnp.zeros((1, 8), F32).at[0, :2].set(c + b2 * jnp.sum(m, axis=1))

    out = _mlp(diff, w1, b1, w2, mt, cfold)
    return out[:p, :2]


# speedup vs baseline: 28.3651x; 1.0649x over previous
import functools

import jax
import jax.numpy as jnp
from jax import lax
from jax.experimental import pallas as pl
from jax.experimental.pallas import tpu as pltpu
from jax.experimental.pallas import tpu_sc as plsc

K_NEAR = 10
NW = 32
F32 = jnp.float32


def _knn_body(q_ref, bt_ref, out_ref):
    q = q_ref[...]
    bt = bt_ref[...]
    aa = jnp.sum(q * q, axis=1, keepdims=True)
    bb = jnp.sum(bt * bt, axis=0, keepdims=True)
    d2 = aa + bb - 2.0 * jnp.dot(q, bt, preferred_element_type=F32)
    d2 = jnp.sqrt(jnp.maximum(d2, 0.0))
    br, n2 = d2.shape
    iota = lax.broadcasted_iota(jnp.int32, (br, n2), 1)
    BIGI = jnp.int32(2**30)
    INF = jnp.float32(3e38)
    sels = []
    for _ in range(K_NEAR):
        m = jnp.min(d2, axis=1, keepdims=True)
        cand = jnp.where(d2 <= m, iota, BIGI)
        sel = jnp.min(cand, axis=1, keepdims=True)
        sels.append(sel)
        d2 = jnp.where(iota == sel, INF, d2)
    for r in range(K_NEAR):
        for i in range(r % 2, K_NEAR - 1, 2):
            lo = jnp.minimum(sels[i], sels[i + 1])
            hi = jnp.maximum(sels[i], sels[i + 1])
            sels[i], sels[i + 1] = lo, hi
    lane = lax.broadcasted_iota(jnp.int32, (br, 128), 1)
    res = jnp.zeros((br, 128), jnp.int32)
    for k in range(K_NEAR):
        res = jnp.where(lane == k, sels[k], res)
    out_ref[...] = res


def _knn(a, b, br=200):
    n1, n2 = a.shape[0], b.shape[0]
    qpad = jnp.zeros((n1, 8), F32).at[:, :3].set(a)
    btpad = jnp.zeros((8, n2), F32).at[:3, :].set(b.T)
    return pl.pallas_call(
        _knn_body,
        grid=(n1 // br,),
        in_specs=[
            pl.BlockSpec((br, 8), lambda i: (i, 0)),
            pl.BlockSpec((8, n2), lambda i: (0, 0)),
        ],
        out_specs=pl.BlockSpec((br, 128), lambda i: (i, 0)),
        out_shape=jax.ShapeDtypeStruct((n1, 128), jnp.int32),
    )(qpad, btpad)


def _dense_first_body(x_ref, w_ref, attm_ref, h_ref, a2_ref):
    x = x_ref[...]
    h = jnp.dot(x, w_ref[...], preferred_element_type=F32)
    h_ref[...] = h
    a2_ref[...] = jnp.dot(h, attm_ref[...], preferred_element_type=F32)


def _dense_first(x_pad, w_pad, attm, bn=1280):
    n_pad = x_pad.shape[0]
    fin = x_pad.shape[1]
    fpad = w_pad.shape[1]
    return pl.pallas_call(
        _dense_first_body,
        grid=(n_pad // bn,),
        in_specs=[
            pl.BlockSpec((bn, fin), lambda i: (i, 0)),
            pl.BlockSpec(w_pad.shape, lambda i: (0, 0)),
            pl.BlockSpec(attm.shape, lambda i: (0, 0)),
        ],
        out_specs=(pl.BlockSpec((bn, fpad), lambda i: (i, 0)),
                   pl.BlockSpec((bn, 8), lambda i: (i, 0))),
        out_shape=(jax.ShapeDtypeStruct((n_pad, fpad), F32),
                   jax.ShapeDtypeStruct((n_pad, 8), F32)),
    )(x_pad, w_pad, attm)


def _dense_mid_body(u_ref, dent_ref, b_ref, w_ref, attm_ref,
                    xl_ref, h_ref, a2_ref):
    den = jnp.sum(dent_ref[...], axis=1, keepdims=True)
    xl = (u_ref[0] + u_ref[1]) / jnp.maximum(den, 1e-30) + b_ref[...]
    xl_ref[...] = xl
    h = jnp.dot(xl, w_ref[...], preferred_element_type=F32)
    h_ref[...] = h
    a2_ref[...] = jnp.dot(h, attm_ref[...], preferred_element_type=F32)


def _dense_mid(u, dent, bias_row, w_pad, attm, bn=1280):
    n_pad = u.shape[1]
    fprev = u.shape[2]
    fpad = w_pad.shape[1]
    nw = dent.shape[1]
    return pl.pallas_call(
        _dense_mid_body,
        grid=(n_pad // bn,),
        in_specs=[
            pl.BlockSpec((2, bn, fprev), lambda i: (0, i, 0)),
            pl.BlockSpec((bn, nw), lambda i: (i, 0)),
            pl.BlockSpec((1, fprev), lambda i: (0, 0)),
            pl.BlockSpec(w_pad.shape, lambda i: (0, 0)),
            pl.BlockSpec(attm.shape, lambda i: (0, 0)),
        ],
        out_specs=(pl.BlockSpec((bn, fprev), lambda i: (i, 0)),
                   pl.BlockSpec((bn, fpad), lambda i: (i, 0)),
                   pl.BlockSpec((bn, 8), lambda i: (i, 0))),
        out_shape=(jax.ShapeDtypeStruct((n_pad, fprev), F32),
                   jax.ShapeDtypeStruct((n_pad, fpad), F32),
                   jax.ShapeDtypeStruct((n_pad, 8), F32)),
    )(u, dent, bias_row, w_pad, attm)


def _gat_agg(packed, asrc, adst, h):
    n_pad, fpad = h.shape
    nchunks = packed.shape[1]
    rpt = n_pad // 16
    mesh = plsc.VectorSubcoreMesh(core_axis_name="c", subcore_axis_name="s")

    @functools.partial(
        pl.kernel,
        out_type=(jax.ShapeDtypeStruct((2, n_pad, fpad), F32),
                  jax.ShapeDtypeStruct((NW, n_pad), F32)),
        mesh=mesh,
        compiler_params=pltpu.CompilerParams(needs_layout_passes=False, use_tc_tiling_on_sc=False),
        scratch_types=[
            pltpu.VMEM((nchunks, 128), jnp.int32),
            pltpu.VMEM((nchunks, 128), jnp.int32),
            pltpu.VMEM((n_pad,), F32),
            pltpu.VMEM((n_pad,), F32),
            pltpu.VMEM((n_pad,), F32),
            pltpu.VMEM((128, fpad), F32),
            pltpu.VMEM((128,), F32),
            pltpu.VMEM((rpt // 8, fpad), F32),
            pltpu.VMEM_SHARED((n_pad, fpad), F32),
            pltpu.SemaphoreType.DMA,
        ],
    )
    def k(packed_hbm, asrc_hbm, adst_hbm, h_hbm, u_out, den_out,
          src_v, dst_v, asrc_v, adst_v, den_loc, rowbuf, exbuf, zbuf,
          u_sh, sem):
        c = lax.axis_index("c")
        s = lax.axis_index("s")
        wid = s * 2 + c
        zero16 = jnp.zeros((16,), F32)
        zrows = rpt // 8

        pltpu.sync_copy(packed_hbm.at[wid], src_v)
        pltpu.sync_copy(asrc_hbm, asrc_v)
        pltpu.sync_copy(adst_hbm, adst_v)

        def unpk(i, carry):
            for j in range(8):
                v = src_v[i, pl.ds(j * 16, 16)]
                src_v[i, pl.ds(j * 16, 16)] = v & jnp.int32(0xFFFF)
                dst_v[i, pl.ds(j * 16, 16)] = lax.shift_right_logical(v, 16)
            return carry
        lax.fori_loop(0, nchunks, unpk, 0)

        def zdl(i, carry):
            den_loc[pl.ds(i * 16, 16)] = zero16
            return carry
        lax.fori_loop(0, n_pad // 16, zdl, 0)

        def zzb(i, carry):
            for cc in range(fpad // 16):
                zbuf[i, pl.ds(cc * 16, 16)] = zero16
            return carry
        lax.fori_loop(0, zrows, zzb, 0)
        for kk in range(8):
            pltpu.sync_copy(zbuf, u_sh.at[pl.ds(s * rpt + kk * zrows, zrows)])
        plsc.subcore_barrier()

        lane16 = lax.broadcasted_iota(jnp.int32, (16,), 0)

        def chunk(i, carry):
            pltpu.async_copy(h_hbm.at[src_v.at[i]], rowbuf, sem).wait()

            def grp(j, carry2):
                s16 = src_v[i, pl.ds(j * 16, 16)]
                d16 = dst_v[i, pl.ds(j * 16, 16)]
                av = (plsc.load_gather(asrc_v, [s16])
                      + plsc.load_gather(adst_v, [d16]))
                al = jnp.where(av >= 0, av, jnp.float32(0.2) * av)
                ex = jnp.exp(al)
                exbuf[pl.ds(j * 16, 16)] = ex
                for l in range(16):
                    plsc.addupdate_scatter(den_loc, [d16], ex,
                                           mask=lane16 == l)
                return carry2
            lax.fori_loop(0, 8, grp, 0)

            def rmul(j, carry2):
                ex16 = exbuf[pl.ds(j * 16, 16)]
                for l in range(16):
                    ce = ex16[l]
                    for cc in range(fpad // 16):
                        e = j * 16 + l
                        rowbuf[e, pl.ds(cc * 16, 16)] = (
                            rowbuf[e, pl.ds(cc * 16, 16)] * ce)
                return carry2
            lax.fori_loop(0, 8, rmul, 0)

            pltpu.sync_copy(rowbuf, u_sh.at[dst_v.at[i]], add=True)
            return carry
        lax.fori_loop(0, nchunks, chunk, 0)
        pltpu.sync_copy(den_loc, den_out.at[wid])
        plsc.subcore_barrier()

        pltpu.sync_copy(u_sh.at[pl.ds(s * rpt, rpt)],
                        u_out.at[c, pl.ds(s * rpt, rpt)])

    return k(packed, asrc, adst, h)


def _match_diff(ridx2d, cidx2d, pred1, pred2):
    nchunks = ridx2d.shape[1]
    ppw = nchunks * 128
    p_pad = NW * ppw
    mesh = plsc.VectorSubcoreMesh(core_axis_name="c", subcore_axis_name="s")

    @functools.partial(
        pl.kernel,
        out_type=jax.ShapeDtypeStruct((p_pad, 32), F32),
        mesh=mesh,
        compiler_params=pltpu.CompilerParams(needs_layout_passes=False, use_tc_tiling_on_sc=False),
        scratch_types=[
            pltpu.VMEM((nchunks, 128), jnp.int32),
            pltpu.VMEM((nchunks, 128), jnp.int32),
            pltpu.VMEM((128, 32), F32),
            pltpu.VMEM((128, 32), F32),
            pltpu.SemaphoreType.DMA,
        ],
    )
    def k(ridx_hbm, cidx_hbm, p1_hbm, p2_hbm, out_hbm, r_v, c_v, b1, b2, sem):
        c = lax.axis_index("c")
        s = lax.axis_index("s")
        wid = s * 2 + c
        pltpu.sync_copy(ridx_hbm.at[wid], r_v)
        pltpu.sync_copy(cidx_hbm.at[wid], c_v)

        def chunk(i, carry):
            pltpu.async_copy(p1_hbm.at[r_v.at[i]], b1, sem).wait()
            pltpu.async_copy(p2_hbm.at[c_v.at[i]], b2, sem).wait()

            def rm(e, carry2):
                for cc in range(2):
                    d = (b1[e, pl.ds(cc * 16, 16)]
                         - b2[e, pl.ds(cc * 16, 16)])
                    b1[e, pl.ds(cc * 16, 16)] = jnp.abs(d)
                return carry2
            lax.fori_loop(0, 128, rm, 0)
            pltpu.sync_copy(b1, out_hbm.at[pl.ds(wid * ppw + i * 128, 128)])
            return carry
        lax.fori_loop(0, nchunks, chunk, 0)

    return k(ridx2d, cidx2d, pred1, pred2)


def _project_body(x_ref, a1_ref, a2_ref, a3_ref, a4_ref, u5_ref, den5_ref,
                  b5_ref, g1_ref, ug2_ref, deng2_ref, bg2_ref,
                  wa_ref, wb_ref, cvec_ref, out_ref):
    den5 = jnp.sum(den5_ref[...], axis=1, keepdims=True)
    a5 = (u5_ref[0] + u5_ref[1]) / jnp.maximum(den5, 1e-30) + b5_ref[...]
    deng2 = jnp.sum(deng2_ref[...], axis=1, keepdims=True)
    g2 = (ug2_ref[0] + ug2_ref[1]) / jnp.maximum(deng2, 1e-30) + bg2_ref[...]
    wa = wa_ref[...]
    wb = wb_ref[...]
    x = x_ref[...]
    emb1 = (jnp.dot(x, wa[0:16], preferred_element_type=F32)
            + jnp.dot(a1_ref[...], wa[16:80], preferred_element_type=F32)
            + jnp.dot(a2_ref[...], wa[80:144], preferred_element_type=F32)
            + jnp.dot(a3_ref[...], wa[144:208], preferred_element_type=F32)
            + jnp.dot(a4_ref[...], wa[208:272], preferred_element_type=F32)
            + jnp.dot(a5, wa[272:336], preferred_element_type=F32))
    emb2 = (jnp.dot(x, wb[0:16], preferred_element_type=F32)
            + jnp.dot(g1_ref[...], wb[16:32], preferred_element_type=F32)
            + jnp.dot(g2, wb[32:48], preferred_element_type=F32))
    out_ref[...] = jnp.concatenate([emb2, emb1], axis=1) + cvec_ref[...]


def _project(x_pad, a_list, u5, den5t, b5row, g1, ug2, deng2t, bg2row,
             wa, wb, cvec, bn=1280):
    n_pad = x_pad.shape[0]
    nw = den5t.shape[1]
    full = lambda arr: pl.BlockSpec(arr.shape, lambda i: tuple(0 for _ in arr.shape))
    row2 = lambda f: pl.BlockSpec((bn, f), lambda i: (i, 0))
    return pl.pallas_call(
        _project_body,
        grid=(n_pad // bn,),
        in_specs=[
            row2(16), row2(64), row2(64), row2(64), row2(64),
            pl.BlockSpec((2, bn, 64), lambda i: (0, i, 0)),
            row2(nw), full(b5row), row2(16),
            pl.BlockSpec((2, bn, 16), lambda i: (0, i, 0)),
            row2(nw), full(bg2row), full(wa), full(wb), full(cvec),
        ],
        out_specs=row2(32),
        out_shape=jax.ShapeDtypeStruct((n_pad, 32), F32),
    )(x_pad, a_list[0], a_list[1], a_list[2], a_list[3], u5, den5t,
      b5row, g1, ug2, deng2t, bg2row, wa, wb, cvec)


BF = jnp.bfloat16


def _mlp_body(d_ref, w1_ref, b1_ref, w2_ref, b2_ref, mwt_ref, mb_ref,
              m2t_ref, c2_ref, out_ref):
    d = d_ref[...]
    acc = jnp.zeros_like(d)
    for j in range(32):
        hj = jnp.maximum(d * w1_ref[0, j] + b1_ref[0, j], 0.0)
        acc = acc + hj * w2_ref[0, j]
    t = acc + b2_ref[0, 0]
    y = jnp.dot(t, mwt_ref[...],
                preferred_element_type=F32) + mb_ref[...]
    out_ref[...] = jnp.dot(y, m2t_ref[...],
                           preferred_element_type=F32) + c2_ref[...]


def _mlp(diff, w1, b1, w2, b2, mwt, mb, m2t, c2, bp=2048):
    p_pad = diff.shape[0]
    return pl.pallas_call(
        _mlp_body,
        grid=(p_pad // bp,),
        in_specs=[
            pl.BlockSpec((bp, 32), lambda i: (i, 0)),
            pl.BlockSpec(memory_space=pltpu.SMEM),
            pl.BlockSpec(memory_space=pltpu.SMEM),
            pl.BlockSpec(memory_space=pltpu.SMEM),
            pl.BlockSpec(memory_space=pltpu.SMEM),
            pl.BlockSpec((32, 16), lambda i: (0, 0)),
            pl.BlockSpec((1, 16), lambda i: (0, 0)),
            pl.BlockSpec((16, 8), lambda i: (0, 0)),
            pl.BlockSpec((1, 8), lambda i: (0, 0)),
        ],
        out_specs=pl.BlockSpec((bp, 8), lambda i: (i, 0)),
        out_shape=jax.ShapeDtypeStruct((p_pad, 8), F32),
    )(diff, w1, b1, w2, b2, mwt, mb, m2t, c2)


def _pad_w(w, fin_pad, fout_pad):
    return jnp.zeros((fin_pad, fout_pad), F32).at[:w.shape[0], :w.shape[1]].set(w)


def _attm(p, fpad):
    m = jnp.zeros((fpad, 8), F32)
    m = m.at[:p["att_src"].shape[0], 0].set(p["att_src"])
    m = m.at[:p["att_dst"].shape[0], 1].set(p["att_dst"])
    return m


def _bias_row(b, fpad):
    return jnp.zeros((1, fpad), F32).at[0, :b.shape[0]].set(b)


def _dent(den, n_pad):
    return den.T


def _fin_pad(f):
    return max(16, -(-f // 16) * 16)


def _run_graph(params, x, edge_index, n):
    n_pad = 10240 if n == 10000 else -(-n // 256) * 256
    e_tot = edge_index.shape[1] + n
    e_pad = -(-e_tot // (NW * 128)) * (NW * 128)
    dummy = n

    loop = jnp.arange(n, dtype=edge_index.dtype)
    fill = jnp.full((e_pad - e_tot,), dummy, jnp.int32)
    src_all = jnp.concatenate([edge_index[0], loop, fill])
    dst_all = jnp.concatenate([edge_index[1], loop, fill])
    packed = (src_all | (dst_all << 16)).reshape(NW, -1, 128)

    x_pad = jnp.zeros((n_pad, 16), F32).at[:n, :x.shape[1]].set(x)

    def run_enc(layers, nlayers, fpad):
        feats = []
        u = den = None
        for li in range(nlayers):
            p = layers[li]
            w = _pad_w(p["W"], _fin_pad(p["W"].shape[0]) if li else 16, fpad)
            am = _attm(p, fpad)
            if li == 0:
                h, a2 = _dense_first(x_pad, w, am)
            else:
                bias = _bias_row(layers[li - 1]["bias"], fpad)
                xl, h, a2 = _dense_mid(u, _dent(den, n_pad), bias, w, am)
                feats.append(xl)
            asrc = a2[:, 0]
            adst = a2[:, 1]
            u, den = _gat_agg(packed, asrc, adst, h)
        return feats, u, den

    feats1, u5, den5 = run_enc(params["enc1"]["layers"], 5, 64)
    feats2, ug2, deng2 = run_enc(params["enc2"]["layers"], 2, 16)

    cw1 = params["enc1"]["conv_w"][:, 0, :]
    cw2 = params["enc2"]["conv_w"][:, 0, :]
    wa = jnp.zeros((336, 16), F32)
    wa = wa.at[0:10].set(cw1[:, 0:10].T)
    for i in range(4):
        wa = wa.at[16 + 64 * i:16 + 64 * i + 64].set(
            cw1[:, 10 + 64 * i:74 + 64 * i].T)
    wa = wa.at[272:272 + 55].set(cw1[:, 266:321].T)
    wb = jnp.zeros((48, 16), F32)
    wb = wb.at[0:10].set(cw2[:, 0:10].T)
    wb = wb.at[16:24].set(cw2[:, 10:18].T)
    wb = wb.at[32:39].set(cw2[:, 18:25].T)
    cvec = jnp.concatenate(
        [params["enc2"]["conv_b"], params["enc1"]["conv_b"]])[None, :]

    b5row = _bias_row(params["enc1"]["layers"][4]["bias"], 64)
    bg2row = _bias_row(params["enc2"]["layers"][1]["bias"], 16)
    return _project(x_pad, feats1, u5, _dent(den5, n_pad), b5row,
                    feats2[0], ug2, _dent(deng2, n_pad), bg2row,
                    wa, wb, cvec)


def kernel(x1, edge_attr1, x2, edge_attr2, params, edge_index1, edge_index2):
    n1 = x1.shape[0]
    n2 = x2.shape[0]
    pred1 = _run_graph(params, x1, edge_index1, n1)
    pred2 = _run_graph(params, x2, edge_index2, n2)

    cols = _knn(x1[:, 0:3], x2[:, 0:3])[:, :K_NEAR].reshape(-1)
    p = n1 * K_NEAR
    p_pad = -(-p // (NW * 128)) * (NW * 128)
    rows = jnp.repeat(jnp.arange(n1, dtype=jnp.int32), K_NEAR)
    ridx2d = jnp.concatenate(
        [rows, jnp.zeros((p_pad - p,), jnp.int32)]).reshape(NW, -1, 128)
    cidx2d = jnp.concatenate(
        [cols, jnp.zeros((p_pad - p,), jnp.int32)]).reshape(NW, -1, 128)

    diff = _match_diff(ridx2d, cidx2d, pred1, pred2)

    mlp = params["mlp"]
    w1 = mlp["W1"][0][None, :]
    b1 = mlp["b1"][None, :]
    w2 = mlp["W2"][:, 0][None, :]
    b2 = mlp["b2"][None, :]
    mwt = jnp.zeros((32, 16), F32).at[:, :10].set(params["m_w"].T)
    mb = jnp.zeros((1, 16), F32).at[0, :10].set(params["m_b"])
    m2t = jnp.zeros((16, 8), F32).at[:10, :2].set(params["m2_w"].T)
    c2 = jnp.zeros((1, 8), F32).at[0, :2].set(params["m2_b"])

    out = _mlp(diff, w1, b1, w2, b2, mwt, mb, m2t, c2)
    return out[:p, :2]
